# baseline (speedup 1.0000x reference)
import functools

import jax
import jax.numpy as jnp
from jax import lax
from jax.experimental import pallas as pl
from jax.experimental.pallas import tpu as pltpu
from jax.experimental.pallas import tpu_sc as plsc

_SEQ = 2048
_D = 2048

_info = plsc.get_sparse_core_info()
_NC = _info.num_cores
_NS = _info.num_subcores
_NW = _NC * _NS
_BPW = _SEQ // _NW
_C0, _C1, _C2 = 16, 32, 16

_mesh = plsc.VectorSubcoreMesh(core_axis_name="c", subcore_axis_name="s")


@functools.partial(
    pl.kernel,
    mesh=_mesh,
    out_type=jax.ShapeDtypeStruct((_SEQ, _D), jnp.float32),
    scratch_types=[
        pltpu.VMEM((_BPW,), jnp.int32),
        pltpu.VMEM((_C0, _D), jnp.float32),
        pltpu.VMEM((_C1, _D), jnp.float32),
    ] + [pltpu.SemaphoreType.DMA] * 4,
)
def _gather_rows(table_hbm, idx_hbm, out_hbm, idx_v, buf0, buf1,
                 gsem0, gsem1, wsem0, wsem1):
    wid = lax.axis_index("s") * _NC + lax.axis_index("c")
    base = wid * _BPW

    pltpu.sync_copy(idx_hbm.at[pl.ds(base, _C0)], idx_v.at[pl.ds(0, _C0)])
    g0 = pltpu.async_copy(
        table_hbm.at[idx_v.at[pl.ds(0, _C0)]], buf0, gsem0)
    pltpu.sync_copy(idx_hbm.at[pl.ds(base + _C0, _BPW - _C0)],
                    idx_v.at[pl.ds(_C0, _BPW - _C0)])
    g1 = pltpu.async_copy(
        table_hbm.at[idx_v.at[pl.ds(_C0, _C1)]], buf1, gsem1)

    g0.wait()
    w0 = pltpu.async_copy(buf0, out_hbm.at[pl.ds(base, _C0)], wsem0)
    w0.wait()
    g2 = pltpu.async_copy(
        table_hbm.at[idx_v.at[pl.ds(_C0 + _C1, _C2)]], buf0, gsem0)

    g1.wait()
    w1 = pltpu.async_copy(buf1, out_hbm.at[pl.ds(base + _C0, _C1)], wsem1)
    g2.wait()
    w2 = pltpu.async_copy(buf0, out_hbm.at[pl.ds(base + _C0 + _C1, _C2)],
                          wsem0)
    w1.wait()
    w2.wait()


def kernel(tokens, table):
    idx = tokens.reshape(-1).astype(jnp.int32)
    return _gather_rows(table, idx)


# speedup vs baseline: 1.0196x; 1.0196x over previous
import functools

import jax
import jax.numpy as jnp
from jax import lax
from jax.experimental import pallas as pl
from jax.experimental.pallas import tpu as pltpu
from jax.experimental.pallas import tpu_sc as plsc

_SEQ = 2048
_D = 2048

_info = plsc.get_sparse_core_info()
_NC = _info.num_cores
_NS = _info.num_subcores
_NW = _NC * _NS
_BPW = _SEQ // _NW
_CHUNK = 16
_NBUF = 3
_NCHUNK = _BPW // _CHUNK

_mesh = plsc.VectorSubcoreMesh(core_axis_name="c", subcore_axis_name="s")


@functools.partial(
    pl.kernel,
    mesh=_mesh,
    out_type=jax.ShapeDtypeStruct((_SEQ, _D), jnp.float32),
    scratch_types=[
        pltpu.VMEM((_BPW,), jnp.int32),
        pltpu.VMEM((_CHUNK, _D), jnp.float32),
        pltpu.VMEM((_CHUNK, _D), jnp.float32),
        pltpu.VMEM((_CHUNK, _D), jnp.float32),
    ] + [pltpu.SemaphoreType.DMA] * 6,
)
def _gather_rows(table_hbm, idx_hbm, out_hbm, idx_v, buf0, buf1, buf2,
                 *sems):
    wid = lax.axis_index("s") * _NC + lax.axis_index("c")
    base = wid * _BPW

    bufs = (buf0, buf1, buf2)
    gsems = sems[:_NBUF]
    wsems = sems[_NBUF:]

    gathers = [None] * _NCHUNK
    writes = [None] * _NCHUNK
    pltpu.sync_copy(idx_hbm.at[pl.ds(base, _CHUNK)],
                    idx_v.at[pl.ds(0, _CHUNK)])
    gathers[0] = pltpu.async_copy(
        table_hbm.at[idx_v.at[pl.ds(0, _CHUNK)]], bufs[0], gsems[0])
    pltpu.sync_copy(idx_hbm.at[pl.ds(base + _CHUNK, _BPW - _CHUNK)],
                    idx_v.at[pl.ds(_CHUNK, _BPW - _CHUNK)])
    for c in range(1, min(_NBUF, _NCHUNK)):
        gathers[c] = pltpu.async_copy(
            table_hbm.at[idx_v.at[pl.ds(c * _CHUNK, _CHUNK)]],
            bufs[c % _NBUF], gsems[c % _NBUF])
    for c in range(_NCHUNK):
        b = c % _NBUF
        gathers[c].wait()
        writes[c] = pltpu.async_copy(
            bufs[b], out_hbm.at[pl.ds(base + c * _CHUNK, _CHUNK)], wsems[b])
        nc = c + _NBUF
        if nc < _NCHUNK:
            writes[nc - _NBUF].wait()
            gathers[nc] = pltpu.async_copy(
                table_hbm.at[idx_v.at[pl.ds(nc * _CHUNK, _CHUNK)]],
                bufs[nc % _NBUF], gsems[nc % _NBUF])
    for c in range(max(_NCHUNK - _NBUF, 0), _NCHUNK):
        writes[c].wait()


def kernel(tokens, table):
    idx = tokens.reshape(-1).astype(jnp.int32)
    return _gather_rows(table, idx)
